# baseline (speedup 1.0000x reference)
import jax
import jax.numpy as jnp
from jax import lax
from jax.experimental import pallas as pl
from jax.experimental.pallas import tpu as pltpu, tpu_sc as plsc

VOCAB = 1000000
EMBED_DIM = 64
BATCH = 16384
FIELDS = 26

NC = 2
NS = 16
NW = NC * NS

B_TOTAL = BATCH * FIELDS
B_PER_W = B_TOTAL // NW
CHUNK = 128
CHUNKS_PER_W = B_PER_W // CHUNK
NBUF = 4
GROUPS = CHUNKS_PER_W // NBUF


def _embed_kernel(idx_hbm, table_hbm, out_hbm, idx_v, bufs, gsems, wsems):
    wid = lax.axis_index("s") * NC + lax.axis_index("c")
    base = wid * B_PER_W
    pltpu.sync_copy(idx_hbm.at[wid], idx_v)

    def group(i, carry):
        gdescs = []
        for b in range(NBUF):
            j = i * NBUF + b
            d = pltpu.async_copy(table_hbm.at[idx_v.at[j]], bufs.at[b],
                                 gsems.at[b])
            gdescs.append(d)
        wdescs = []
        for b in range(NBUF):
            j = i * NBUF + b
            gdescs[b].wait()
            d = pltpu.async_copy(bufs.at[b],
                                 out_hbm.at[pl.ds(base + j * CHUNK, CHUNK)],
                                 wsems.at[b])
            wdescs.append(d)
        for b in range(NBUF):
            wdescs[b].wait()
        return carry

    lax.fori_loop(0, GROUPS, group, 0)


def kernel(embed_input, weight):
    idx = embed_input.astype(jnp.int32).reshape(NW, CHUNKS_PER_W, CHUNK)
    mesh = plsc.VectorSubcoreMesh(core_axis_name="c", subcore_axis_name="s")
    out = pl.kernel(
        _embed_kernel,
        out_type=jax.ShapeDtypeStruct((B_TOTAL, EMBED_DIM), jnp.float32),
        mesh=mesh,
        compiler_params=pltpu.CompilerParams(use_tc_tiling_on_sc=False),
        scratch_types=[
            pltpu.VMEM((CHUNKS_PER_W, CHUNK), jnp.int32),
            pltpu.VMEM((NBUF, CHUNK, EMBED_DIM), jnp.float32),
            pltpu.SemaphoreType.DMA((NBUF,)),
            pltpu.SemaphoreType.DMA((NBUF,)),
        ],
    )(idx, weight)
    return out.reshape(BATCH, FIELDS, EMBED_DIM)


# speedup vs baseline: 1.0046x; 1.0046x over previous
import jax
import jax.numpy as jnp
from jax import lax
from jax.experimental import pallas as pl
from jax.experimental.pallas import tpu as pltpu, tpu_sc as plsc

VOCAB = 1000000
EMBED_DIM = 64
BATCH = 16384
FIELDS = 26

NC = 2
NS = 16
NW = NC * NS

B_TOTAL = BATCH * FIELDS
B_PER_W = B_TOTAL // NW
CHUNK = 512
CHUNKS_PER_W = B_PER_W // CHUNK
NBUF = 2
GROUPS = CHUNKS_PER_W // NBUF


def _embed_kernel(idx_hbm, table_hbm, out_hbm, idx_v, bufs, gsems, wsems):
    wid = lax.axis_index("s") * NC + lax.axis_index("c")
    base = wid * B_PER_W
    pltpu.sync_copy(idx_hbm.at[wid], idx_v)

    def group(i, carry):
        gdescs = []
        for b in range(NBUF):
            j = i * NBUF + b
            d = pltpu.async_copy(table_hbm.at[idx_v.at[j]], bufs.at[b],
                                 gsems.at[b])
            gdescs.append(d)
        wdescs = []
        for b in range(NBUF):
            j = i * NBUF + b
            gdescs[b].wait()
            d = pltpu.async_copy(bufs.at[b],
                                 out_hbm.at[pl.ds(base + j * CHUNK, CHUNK)],
                                 wsems.at[b])
            wdescs.append(d)
        for b in range(NBUF):
            wdescs[b].wait()
        return carry

    lax.fori_loop(0, GROUPS, group, 0)


def kernel(embed_input, weight):
    idx = embed_input.astype(jnp.int32).reshape(NW, CHUNKS_PER_W, CHUNK)
    mesh = plsc.VectorSubcoreMesh(core_axis_name="c", subcore_axis_name="s")
    out = pl.kernel(
        _embed_kernel,
        out_type=jax.ShapeDtypeStruct((B_TOTAL, EMBED_DIM), jnp.float32),
        mesh=mesh,
        compiler_params=pltpu.CompilerParams(use_tc_tiling_on_sc=False),
        scratch_types=[
            pltpu.VMEM((CHUNKS_PER_W, CHUNK), jnp.int32),
            pltpu.VMEM((NBUF, CHUNK, EMBED_DIM), jnp.float32),
            pltpu.SemaphoreType.DMA((NBUF,)),
            pltpu.SemaphoreType.DMA((NBUF,)),
        ],
    )(idx, weight)
    return out.reshape(BATCH, FIELDS, EMBED_DIM)
